# baseline (speedup 1.0000x reference)
import functools

import jax
import jax.numpy as jnp
from jax import lax
from jax.experimental import pallas as pl
from jax.experimental.pallas import tpu as pltpu
from jax.experimental.pallas import tpu_sc as plsc

VOCAB = 1000000
B = 4096
H = 200
HP = 256
CHUNK = 128
SEGS = HP // CHUNK
D = 32
C = 64
NBUF = 4
TW = 128

_info = plsc.get_sparse_core_info()
NC, NS = _info.num_cores, _info.num_subcores
NW = NC * NS
BPW = B // NW
CPW = BPW * SEGS

_mesh = plsc.VectorSubcoreMesh(core_axis_name="c", subcore_axis_name="s")


@functools.partial(
    pl.kernel,
    mesh=_mesh,
    compiler_params=pltpu.CompilerParams(use_tc_tiling_on_sc=False),
    out_type=jax.ShapeDtypeStruct((B, TW), jnp.float32),
    scratch_types=(
        [
            pltpu.VMEM((CPW, CHUNK), jnp.int32),
            pltpu.VMEM((BPW, TW), jnp.float32),
        ]
        + [pltpu.VMEM((CHUNK, TW), jnp.float32) for _ in range(NBUF)]
        + [pltpu.SemaphoreType.DMA for _ in range(NBUF)]
    ),
)
def _sc_gather_sum(x2_hbm, table_hbm, sums_hbm,
                   idx_v, acc_v, b0, b1, b2, b3, s0, s1, s2, s3):
    bufs = (b0, b1, b2, b3)
    sems = (s0, s1, s2, s3)
    wid = lax.axis_index("s") * NC + lax.axis_index("c")

    pltpu.sync_copy(x2_hbm.at[pl.ds(wid * CPW, CPW)], idx_v)

    for k in range(NBUF):
        pltpu.async_copy(
            table_hbm.at[idx_v.at[k]], bufs[k], sems[k]
        )

    def accum_chunk(slot, c, acc0, acc1):
        pltpu.make_async_copy(
            table_hbm.at[idx_v.at[c]], bufs[slot], sems[slot]
        ).wait()
        for r in range(CHUNK):
            acc0 = acc0 + bufs[slot][r, pl.ds(0, 16)]
            acc1 = acc1 + bufs[slot][r, pl.ds(16, 16)]

        @pl.when(c + NBUF < CPW)
        def _():
            pltpu.async_copy(
                table_hbm.at[idx_v.at[c + NBUF]], bufs[slot], sems[slot]
            )

        return acc0, acc1

    def step(o, carry):
        for p in range(2):
            b = 2 * o + p
            zero = jnp.zeros((16,), jnp.float32)
            acc0, acc1 = zero, zero
            for i in range(SEGS):
                slot = 2 * p + i
                acc0, acc1 = accum_chunk(slot, SEGS * b + i, acc0, acc1)
            acc_v[b, pl.ds(0, 16)] = acc0
            acc_v[b, pl.ds(16, 16)] = acc1
        return carry

    lax.fori_loop(0, BPW // 2, step, 0)

    pltpu.sync_copy(acc_v, sums_hbm.at[pl.ds(wid * BPW, BPW)])


_PB = 8000


def _tc_pad_body(t_ref, o_ref):
    blk = t_ref[...]
    o_ref[...] = jnp.concatenate(
        [blk, jnp.zeros((blk.shape[0], TW - D), jnp.float32)], axis=1
    )


_TCB = 512


def _tc_head(x_ref, sums_ref, w_ref, b_ref, out_ref):
    cnt = jnp.sum((x_ref[...] != 0).astype(jnp.float32), axis=1, keepdims=True)
    avg = sums_ref[:, :D] / jnp.maximum(cnt, 1.0)
    out_ref[...] = (
        jnp.dot(avg, w_ref[...], preferred_element_type=jnp.float32) + b_ref[...]
    )


def kernel(x, emb_table, fc_w, fc_b):
    x = x.astype(jnp.int32)
    x2p = jnp.pad(x, ((0, 0), (0, HP - H))).reshape(B * SEGS, CHUNK)
    table_pad = pl.pallas_call(
        _tc_pad_body,
        grid=(VOCAB // _PB,),
        in_specs=[pl.BlockSpec((_PB, D), lambda i: (i, 0))],
        out_specs=pl.BlockSpec((_PB, TW), lambda i: (i, 0)),
        out_shape=jax.ShapeDtypeStruct((VOCAB, TW), jnp.float32),
    )(emb_table)
    sums = _sc_gather_sum(x2p, table_pad)
    return pl.pallas_call(
        _tc_head,
        grid=(B // _TCB,),
        in_specs=[
            pl.BlockSpec((_TCB, H), lambda i: (i, 0)),
            pl.BlockSpec((_TCB, TW), lambda i: (i, 0)),
            pl.BlockSpec((D, C), lambda i: (0, 0)),
            pl.BlockSpec((1, C), lambda i: (0, 0)),
        ],
        out_specs=pl.BlockSpec((_TCB, C), lambda i: (i, 0)),
        out_shape=jax.ShapeDtypeStruct((B, C), jnp.float32),
    )(x, sums, fc_w, fc_b.reshape(1, C))


# speedup vs baseline: 10.8209x; 10.8209x over previous
import functools

import jax
import jax.numpy as jnp
from jax import lax
from jax.experimental import pallas as pl
from jax.experimental.pallas import tpu as pltpu
from jax.experimental.pallas import tpu_sc as plsc

B = 4096
H = 200
HP = 208
CHUNK = 104
SEGS = HP // CHUNK
D = 32
C = 64
NBUF = 4

_info = plsc.get_sparse_core_info()
NC, NS = _info.num_cores, _info.num_subcores
NW = NC * NS
BPW = B // NW
CPW = BPW * SEGS

_mesh = plsc.VectorSubcoreMesh(core_axis_name="c", subcore_axis_name="s")


@functools.partial(
    pl.kernel,
    mesh=_mesh,
    compiler_params=pltpu.CompilerParams(use_tc_tiling_on_sc=False),
    out_type=jax.ShapeDtypeStruct((B, D), jnp.float32),
    scratch_types=(
        [
            pltpu.VMEM((CPW, CHUNK), jnp.int32),
            pltpu.VMEM((BPW, D), jnp.float32),
        ]
        + [pltpu.VMEM((CHUNK, D), jnp.float32) for _ in range(NBUF)]
        + [pltpu.SemaphoreType.DMA for _ in range(NBUF)]
    ),
)
def _sc_gather_sum(x2_hbm, table_hbm, sums_hbm,
                   idx_v, acc_v, b0, b1, b2, b3, s0, s1, s2, s3):
    bufs = (b0, b1, b2, b3)
    sems = (s0, s1, s2, s3)
    wid = lax.axis_index("s") * NC + lax.axis_index("c")

    pltpu.sync_copy(x2_hbm.at[pl.ds(wid * CPW, CPW)], idx_v)

    for k in range(NBUF):
        pltpu.async_copy(table_hbm.at[idx_v.at[k]], bufs[k], sems[k])

    def accum_chunk(slot, c, acc0, acc1):
        pltpu.make_async_copy(
            table_hbm.at[idx_v.at[c]], bufs[slot], sems[slot]
        ).wait()
        for r in range(CHUNK):
            acc0 = acc0 + bufs[slot][r, pl.ds(0, 16)]
            acc1 = acc1 + bufs[slot][r, pl.ds(16, 16)]

        @pl.when(c + NBUF < CPW)
        def _():
            pltpu.async_copy(
                table_hbm.at[idx_v.at[c + NBUF]], bufs[slot], sems[slot]
            )

        return acc0, acc1

    def step(o, carry):
        for p in range(2):
            b = 2 * o + p
            zero = jnp.zeros((16,), jnp.float32)
            acc0, acc1 = zero, zero
            for i in range(SEGS):
                slot = 2 * p + i
                acc0, acc1 = accum_chunk(slot, SEGS * b + i, acc0, acc1)
            acc_v[b, pl.ds(0, 16)] = acc0
            acc_v[b, pl.ds(16, 16)] = acc1
        return carry

    lax.fori_loop(0, BPW // 2, step, 0)

    pltpu.sync_copy(acc_v, sums_hbm.at[pl.ds(wid * BPW, BPW)])


_TCB = 512


def _tc_head(x_ref, sums_ref, w_ref, b_ref, out_ref):
    cnt = jnp.sum((x_ref[...] != 0).astype(jnp.float32), axis=1, keepdims=True)
    avg = sums_ref[...] / jnp.maximum(cnt, 1.0)
    out_ref[...] = (
        jnp.dot(avg, w_ref[...], preferred_element_type=jnp.float32) + b_ref[...]
    )


def kernel(x, emb_table, fc_w, fc_b):
    x = x.astype(jnp.int32)
    x2 = jnp.pad(x, ((0, 0), (0, HP - H))).reshape(B * SEGS, CHUNK)
    sums = _sc_gather_sum(x2, emb_table)
    return pl.pallas_call(
        _tc_head,
        grid=(B // _TCB,),
        in_specs=[
            pl.BlockSpec((_TCB, H), lambda i: (i, 0)),
            pl.BlockSpec((_TCB, D), lambda i: (i, 0)),
            pl.BlockSpec((D, C), lambda i: (0, 0)),
            pl.BlockSpec((1, C), lambda i: (0, 0)),
        ],
        out_specs=pl.BlockSpec((_TCB, C), lambda i: (i, 0)),
        out_shape=jax.ShapeDtypeStruct((B, C), jnp.float32),
    )(x, sums, fc_w, fc_b.reshape(1, C))


# speedup vs baseline: 16.3971x; 1.5153x over previous
import functools

import jax
import jax.numpy as jnp
from jax import lax
from jax.experimental import pallas as pl
from jax.experimental.pallas import tpu as pltpu
from jax.experimental.pallas import tpu_sc as plsc

B = 4096
H = 200
CA = 104
CB = H - CA
D = 32
C = 64
NBUF = 4

_info = plsc.get_sparse_core_info()
NC, NS = _info.num_cores, _info.num_subcores
NW = NC * NS
BPW = B // NW

_mesh = plsc.VectorSubcoreMesh(core_axis_name="c", subcore_axis_name="s")


@functools.partial(
    pl.kernel,
    mesh=_mesh,
    compiler_params=pltpu.CompilerParams(use_tc_tiling_on_sc=False),
    out_type=jax.ShapeDtypeStruct((B, D), jnp.float32),
    scratch_types=(
        [
            pltpu.VMEM((BPW, CA), jnp.int32),
            pltpu.VMEM((BPW, CB), jnp.int32),
            pltpu.VMEM((BPW, D), jnp.float32),
        ]
        + [pltpu.VMEM((CA, D), jnp.float32), pltpu.VMEM((CB, D), jnp.float32)]
        * (NBUF // 2)
        + [pltpu.SemaphoreType.DMA for _ in range(NBUF)]
    ),
)
def _sc_gather_sum(xa_hbm, xb_hbm, table_hbm, sums_hbm,
                   idxa_v, idxb_v, acc_v, b0, b1, b2, b3, s0, s1, s2, s3):
    bufs = (b0, b1, b2, b3)
    sems = (s0, s1, s2, s3)
    idxs = (idxa_v, idxb_v)
    nrows = (CA, CB)
    wid = lax.axis_index("s") * NC + lax.axis_index("c")

    pltpu.sync_copy(xa_hbm.at[pl.ds(wid * BPW, BPW)], idxa_v)
    pltpu.sync_copy(xb_hbm.at[pl.ds(wid * BPW, BPW)], idxb_v)

    for k in range(NBUF):
        pltpu.async_copy(
            table_hbm.at[idxs[k % 2].at[k // 2]], bufs[k], sems[k]
        )

    def accum_chunk(slot, b, acc0, acc1):
        half = slot % 2
        pltpu.make_async_copy(
            table_hbm.at[idxs[half].at[b]], bufs[slot], sems[slot]
        ).wait()
        for r in range(nrows[half]):
            acc0 = acc0 + bufs[slot][r, pl.ds(0, 16)]
            acc1 = acc1 + bufs[slot][r, pl.ds(16, 16)]

        @pl.when(b + 2 < BPW)
        def _():
            pltpu.async_copy(
                table_hbm.at[idxs[half].at[b + 2]], bufs[slot], sems[slot]
            )

        return acc0, acc1

    def step(o, carry):
        for p in range(2):
            b = 2 * o + p
            zero = jnp.zeros((16,), jnp.float32)
            acc0, acc1 = zero, zero
            for half in range(2):
                slot = 2 * p + half
                acc0, acc1 = accum_chunk(slot, b, acc0, acc1)
            acc_v[b, pl.ds(0, 16)] = acc0
            acc_v[b, pl.ds(16, 16)] = acc1
        return carry

    lax.fori_loop(0, BPW // 2, step, 0)

    pltpu.sync_copy(acc_v, sums_hbm.at[pl.ds(wid * BPW, BPW)])


_TCB = 512


def _tc_head(x_ref, sums_ref, w_ref, b_ref, out_ref):
    cnt = jnp.sum((x_ref[...] != 0).astype(jnp.float32), axis=1, keepdims=True)
    avg = sums_ref[...] / jnp.maximum(cnt, 1.0)
    out_ref[...] = (
        jnp.dot(avg, w_ref[...], preferred_element_type=jnp.float32) + b_ref[...]
    )


def kernel(x, emb_table, fc_w, fc_b):
    x = x.astype(jnp.int32)
    sums = _sc_gather_sum(x[:, :CA], x[:, CA:], emb_table)
    return pl.pallas_call(
        _tc_head,
        grid=(B // _TCB,),
        in_specs=[
            pl.BlockSpec((_TCB, H), lambda i: (i, 0)),
            pl.BlockSpec((_TCB, D), lambda i: (i, 0)),
            pl.BlockSpec((D, C), lambda i: (0, 0)),
            pl.BlockSpec((1, C), lambda i: (0, 0)),
        ],
        out_specs=pl.BlockSpec((_TCB, C), lambda i: (i, 0)),
        out_shape=jax.ShapeDtypeStruct((B, C), jnp.float32),
    )(x, sums, fc_w, fc_b.reshape(1, C))
